# Initial kernel scaffold; baseline (speedup 1.0000x reference)
import jax
import jax.numpy as jnp
from jax.experimental import pallas as pl


def kernel(input_tensor, grid):
    raise NotImplementedError("write your pallas kernel here")


# speedup vs baseline: 7.1606x; 7.1606x over previous
import functools

import jax
import jax.numpy as jnp
from jax import lax
from jax.experimental import pallas as pl
from jax.experimental.pallas import tpu as pltpu
from jax.experimental.pallas import tpu_sc as plsc

_C = 128
_H = 128
_W = 128
_HW = _H * _W
_HG = 7
_WG = 25281
_P = _HG * _WG
_B = 2048
_NCHUNK = -(-_P // _B)
_P_PAD = _NCHUNK * _B
_NTILE = 32
_CPT = _C // _NTILE
_NG = _B // 16


def _sc_grid_sample(planes, gx, gy):
  mesh = plsc.VectorSubcoreMesh(core_axis_name="c", subcore_axis_name="s")

  @functools.partial(
      pl.kernel,
      out_type=jax.ShapeDtypeStruct((_C, _P_PAD), jnp.float32),
      mesh=mesh,
      compiler_params=pltpu.CompilerParams(needs_layout_passes=False),
      scratch_types=[
          pltpu.VMEM((_CPT * _HW,), jnp.float32),
          pltpu.VMEM((_B,), jnp.float32),
          pltpu.VMEM((_B,), jnp.float32),
          pltpu.VMEM((_CPT, _B), jnp.float32),
      ],
  )
  def k(planes_hbm, gx_hbm, gy_hbm, out_hbm, plane_v, gx_v, gy_v, out_v):
    wid = lax.axis_index("c") * 16 + lax.axis_index("s")
    c0 = wid * _CPT
    pltpu.sync_copy(planes_hbm.at[pl.ds(c0 * _HW, _CPT * _HW)], plane_v)

    def chunk_body(ci, carry):
      base = ci * _B
      pltpu.sync_copy(gx_hbm.at[pl.ds(base, _B)], gx_v)
      pltpu.sync_copy(gy_hbm.at[pl.ds(base, _B)], gy_v)

      def group_body(g, gcarry):
        off = g * 16
        gx16 = gx_v[pl.ds(off, 16)]
        gy16 = gy_v[pl.ds(off, 16)]
        ix = ((gx16 + 1.0) * 128.0 - 1.0) * 0.5
        iy = ((gy16 + 1.0) * 128.0 - 1.0) * 0.5
        ix = jnp.minimum(jnp.maximum(ix, -2.0), 129.0)
        iy = jnp.minimum(jnp.maximum(iy, -2.0), 129.0)
        tx = ix.astype(jnp.int32).astype(jnp.float32)
        ty = iy.astype(jnp.int32).astype(jnp.float32)
        fx0 = jnp.where(tx > ix, tx - 1.0, tx)
        fy0 = jnp.where(ty > iy, ty - 1.0, ty)
        fx1 = fx0 + 1.0
        fy1 = fy0 + 1.0
        wx1 = ix - fx0
        wx0 = 1.0 - wx1
        wy1 = iy - fy0
        wy0 = 1.0 - wy1
        vx0 = (fx0 >= 0.0) & (fx0 <= 127.0)
        vx1 = (fx1 >= 0.0) & (fx1 <= 127.0)
        vy0 = (fy0 >= 0.0) & (fy0 <= 127.0)
        vy1 = (fy1 >= 0.0) & (fy1 <= 127.0)
        zero = jnp.zeros((16,), jnp.float32)
        w00 = jnp.where(vx0 & vy0, wx0 * wy0, zero)
        w01 = jnp.where(vx1 & vy0, wx1 * wy0, zero)
        w10 = jnp.where(vx0 & vy1, wx0 * wy1, zero)
        w11 = jnp.where(vx1 & vy1, wx1 * wy1, zero)
        x0 = jnp.minimum(jnp.maximum(fx0, 0.0), 127.0).astype(jnp.int32)
        x1 = jnp.minimum(jnp.maximum(fx1, 0.0), 127.0).astype(jnp.int32)
        y0 = jnp.minimum(jnp.maximum(fy0, 0.0), 127.0).astype(jnp.int32)
        y1 = jnp.minimum(jnp.maximum(fy1, 0.0), 127.0).astype(jnp.int32)
        i00 = y0 * _W + x0
        i01 = y0 * _W + x1
        i10 = y1 * _W + x0
        i11 = y1 * _W + x1
        for c in range(_CPT):
          cb = jnp.full((16,), c * _HW, jnp.int32)
          v00 = plsc.load_gather(plane_v, [cb + i00])
          v01 = plsc.load_gather(plane_v, [cb + i01])
          v10 = plsc.load_gather(plane_v, [cb + i10])
          v11 = plsc.load_gather(plane_v, [cb + i11])
          acc = v00 * w00 + v01 * w01 + v10 * w10 + v11 * w11
          out_v[c, pl.ds(off, 16)] = acc
        return gcarry

      lax.fori_loop(0, _NG, group_body, 0)
      for c in range(_CPT):
        pltpu.sync_copy(out_v.at[c], out_hbm.at[c0 + c, pl.ds(base, _B)])
      return carry

    lax.fori_loop(0, _NCHUNK, chunk_body, 0)

  return k(planes, gx, gy)


def kernel(input_tensor, grid):
  planes = input_tensor.reshape(_C * _HW)
  g = grid.reshape(_P, 2)
  gx = jnp.pad(g[:, 0], (0, _P_PAD - _P))
  gy = jnp.pad(g[:, 1], (0, _P_PAD - _P))
  out = _sc_grid_sample(planes, gx, gy)
  return out[:, :_P].reshape(1, _C, _HG, _WG)
